# baseline (speedup 1.0000x reference)
import functools

import jax
import jax.numpy as jnp
from jax import lax
from jax.experimental import pallas as pl
from jax.experimental.pallas import tpu as pltpu
from jax.experimental.pallas import tpu_sc as plsc

N = 10000
E = 320000
IN = 128
DIM = 256
OUT = 128
G = 64

NP = 10240
EPAD = 327680
ER = EPAD // 128
BR = 256
NRB = NP // BR
HALF = DIM // 2
NT = 16
RPT = NP // NT
EPT = EPAD // NT
NBT = EPT // 128
NBD = EPAD // 32 // 128
CHB = 16

_mesh = plsc.VectorSubcoreMesh(
    core_axis_name="c", subcore_axis_name="s", num_cores=2, num_subcores=NT)

_f32 = jnp.float32
_i32 = jnp.int32


def _zero_vmem2d(ref, nrows):
  z16 = jnp.zeros((16,), _f32)
  @pl.loop(0, nrows)
  def _(i):
    for j in range(8):
      ref[i, pl.ds(j * 16, 16)] = z16


def _sc_degree(dst2d):
  @functools.partial(
      pl.kernel,
      out_type=jax.ShapeDtypeStruct((2, NP), _f32),
      mesh=_mesh,
      scratch_types=[
          pltpu.VMEM((NBD, 128), _i32),
          pltpu.VMEM((128,), _f32),
          pltpu.VMEM((RPT,), _f32),
          pltpu.VMEM_SHARED((NP,), _f32),
      ],
  )
  def deg_kernel(dst_hbm, out_hbm, didx, ones_v, zb, deg_sh):
    c = lax.axis_index("c")
    s = lax.axis_index("s")
    z16 = jnp.zeros((16,), _f32)
    @pl.loop(0, RPT // 16)
    def _(j):
      zb[pl.ds(j * 16, 16)] = z16
    o16 = jnp.ones((16,), _f32)
    @pl.loop(0, 8)
    def _(j):
      ones_v[pl.ds(j * 16, 16)] = o16
    sync = pltpu.sync_copy
    sync(zb, deg_sh.at[pl.ds(s * RPT, RPT)])
    sync(dst_hbm.at[pl.ds((c * NT + s) * NBD, NBD)], didx)
    plsc.subcore_barrier()
    @pl.loop(0, NBD)
    def _(b):
      sync(ones_v, deg_sh.at[didx.at[b]], add=True)
    plsc.subcore_barrier()
    sync(deg_sh.at[pl.ds(s * RPT, RPT)], zb)
    sync(zb, out_hbm.at[c, pl.ds(s * RPT, RPT)])

  return deg_kernel(dst2d)


def _sc_edge_pass(gcat, src2d, dst2d):
  @functools.partial(
      pl.kernel,
      out_type=jax.ShapeDtypeStruct((2 * NP, HALF), _f32),
      mesh=_mesh,
      scratch_types=[
          pltpu.VMEM((CHB, 128), _i32),
          pltpu.VMEM((CHB, 128), _i32),
          pltpu.VMEM((128, HALF), _f32),
          pltpu.VMEM((128, HALF), _f32),
          pltpu.VMEM_SHARED((NP, HALF), _f32),
          pltpu.SemaphoreType.DMA,
          pltpu.SemaphoreType.DMA,
          pltpu.SemaphoreType.DMA,
          pltpu.SemaphoreType.DMA,
      ],
  )
  def edge_kernel(g_hbm, src_hbm, dst_hbm, out_hbm, sidx, didx, rows_a,
                  rows_b, acc_sh, gsem_a, gsem_b, ssem_a, ssem_b):
    c = lax.axis_index("c")
    s = lax.axis_index("s")
    sync = pltpu.sync_copy
    _zero_vmem2d(rows_a, 128)
    @pl.loop(0, 5)
    def _(k):
      sync(rows_a, acc_sh.at[pl.ds(s * RPT + k * 128, 128)])
    plsc.subcore_barrier()
    bufs = (rows_a, rows_b)
    gsems = (gsem_a, gsem_b)
    ssems = (ssem_a, ssem_b)
    @pl.loop(0, NBT // CHB)
    def _(ch):
      sync(src_hbm.at[pl.ds(c * ER + s * NBT + ch * CHB, CHB)], sidx)
      sync(dst_hbm.at[pl.ds(s * NBT + ch * CHB, CHB)], didx)
      pltpu.async_copy(g_hbm.at[sidx.at[0]], bufs[0], gsems[0])
      for b in range(CHB):
        k = b % 2
        if b + 1 < CHB:
          if b >= 1:
            pltpu.make_async_copy(
                bufs[1 - k], acc_sh.at[didx.at[b - 1]], ssems[1 - k]).wait()
          pltpu.async_copy(g_hbm.at[sidx.at[b + 1]], bufs[1 - k], gsems[1 - k])
        pltpu.make_async_copy(g_hbm.at[sidx.at[b]], bufs[k], gsems[k]).wait()
        pltpu.async_copy(bufs[k], acc_sh.at[didx.at[b]], ssems[k], add=True)
      pltpu.make_async_copy(
          bufs[0], acc_sh.at[didx.at[CHB - 2]], ssems[0]).wait()
      pltpu.make_async_copy(
          bufs[1], acc_sh.at[didx.at[CHB - 1]], ssems[1]).wait()
    plsc.subcore_barrier()
    sync(acc_sh.at[pl.ds(s * RPT, RPT)],
         out_hbm.at[pl.ds(c * NP + s * RPT, RPT)])

  return edge_kernel(gcat, src2d, dst2d)


_DN11 = (((1,), (1,)), ((), ()))
_DN00 = (((0,), (0,)), ((), ()))
_PREC = lax.Precision.HIGHEST


def _dinv_of(deg_blk):
  d = deg_blk[0, :] + deg_blk[1, :] + 1.0
  return lax.rsqrt(d).reshape(BR, 1)


def _tc_g1(x_p, deg2, W1):
  def body(x_ref, deg_ref, w_ref, out_ref):
    dinv = _dinv_of(deg_ref[...])
    hw = lax.dot_general(x_ref[...], w_ref[...], _DN11,
                         preferred_element_type=_f32, precision=_PREC)
    out_ref[...] = dinv * hw

  return pl.pallas_call(
      body,
      grid=(2, NRB),
      in_specs=[
          pl.BlockSpec((BR, IN), lambda h, r: (r, 0)),
          pl.BlockSpec((2, BR), lambda h, r: (0, r)),
          pl.BlockSpec((HALF, IN), lambda h, r: (h, 0)),
      ],
      out_specs=pl.BlockSpec((BR, HALF), lambda h, r: (h * NRB + r, 0)),
      out_shape=jax.ShapeDtypeStruct((2 * NP, HALF), _f32),
  )(x_p, deg2, W1)


def _tc_mid(acc, g_prev, deg2, b_prev, W):
  def body(al_ref, ah_ref, gl_ref, gh_ref, deg_ref, b_ref, w_ref, out_ref):
    dinv = _dinv_of(deg_ref[...])
    b = b_ref[...]
    h_lo = jnp.maximum(dinv * (al_ref[...] + gl_ref[...]) + b[:HALF], 0.0)
    h_hi = jnp.maximum(dinv * (ah_ref[...] + gh_ref[...]) + b[HALF:], 0.0)
    w = w_ref[...]
    hw = (lax.dot_general(h_lo, w[:, :HALF], _DN11,
                          preferred_element_type=_f32, precision=_PREC) +
          lax.dot_general(h_hi, w[:, HALF:], _DN11,
                          preferred_element_type=_f32, precision=_PREC))
    out_ref[...] = dinv * hw

  return pl.pallas_call(
      body,
      grid=(2, NRB),
      in_specs=[
          pl.BlockSpec((BR, HALF), lambda h, r: (r, 0)),
          pl.BlockSpec((BR, HALF), lambda h, r: (NRB + r, 0)),
          pl.BlockSpec((BR, HALF), lambda h, r: (r, 0)),
          pl.BlockSpec((BR, HALF), lambda h, r: (NRB + r, 0)),
          pl.BlockSpec((2, BR), lambda h, r: (0, r)),
          pl.BlockSpec((DIM,), lambda h, r: (0,)),
          pl.BlockSpec((HALF, DIM), lambda h, r: (h, 0)),
      ],
      out_specs=pl.BlockSpec((BR, HALF), lambda h, r: (h * NRB + r, 0)),
      out_shape=jax.ShapeDtypeStruct((2 * NP, HALF), _f32),
  )(acc, acc, g_prev, g_prev, deg2, b_prev, W)


def _tc_final(acc, g3, deg2, b3, batch_p, lW1, lb1, lW2, lb2):
  def body(al_ref, ah_ref, gl_ref, gh_ref, deg_ref, b_ref, bat_ref,
           lw1_ref, lb1_ref, lw2_ref, lb2_ref, out_ref, s_acc, c_acc):
    r = pl.program_id(0)
    dinv = _dinv_of(deg_ref[...])
    b = b_ref[...]
    h_lo = dinv * (al_ref[...] + gl_ref[...]) + b[:HALF]
    h_hi = dinv * (ah_ref[...] + gh_ref[...]) + b[HALF:]
    bat = bat_ref[...]
    gids = lax.broadcasted_iota(_i32, (BR, G), 1)
    onehot = (bat.reshape(BR, 1) == gids).astype(_f32)
    s_lo = lax.dot_general(onehot, h_lo, _DN00,
                           preferred_element_type=_f32, precision=_PREC)
    s_hi = lax.dot_general(onehot, h_hi, _DN00,
                           preferred_element_type=_f32, precision=_PREC)
    cnt = jnp.broadcast_to(jnp.sum(onehot, axis=0).reshape(G, 1), (G, HALF))

    @pl.when(r == 0)
    def _():
      s_acc[:, :HALF] = s_lo
      s_acc[:, HALF:] = s_hi
      c_acc[...] = cnt

    @pl.when(r > 0)
    def _():
      s_acc[:, :HALF] += s_lo
      s_acc[:, HALF:] += s_hi
      c_acc[...] += cnt

    @pl.when(r == NRB - 1)
    def _():
      pooled = s_acc[...] / jnp.maximum(c_acc[:, :1], 1.0)
      z = jnp.maximum(
          lax.dot_general(pooled, lw1_ref[...], _DN11,
                          preferred_element_type=_f32, precision=_PREC)
          + lb1_ref[...], 0.0)
      out_ref[...] = lax.dot_general(
          z, lw2_ref[...], _DN11,
          preferred_element_type=_f32, precision=_PREC) + lb2_ref[...]

  return pl.pallas_call(
      body,
      grid=(NRB,),
      in_specs=[
          pl.BlockSpec((BR, HALF), lambda r: (r, 0)),
          pl.BlockSpec((BR, HALF), lambda r: (NRB + r, 0)),
          pl.BlockSpec((BR, HALF), lambda r: (r, 0)),
          pl.BlockSpec((BR, HALF), lambda r: (NRB + r, 0)),
          pl.BlockSpec((2, BR), lambda r: (0, r)),
          pl.BlockSpec((DIM,), lambda r: (0,)),
          pl.BlockSpec((BR,), lambda r: (r,)),
          pl.BlockSpec((DIM, DIM), lambda r: (0, 0)),
          pl.BlockSpec((DIM,), lambda r: (0,)),
          pl.BlockSpec((OUT, DIM), lambda r: (0, 0)),
          pl.BlockSpec((OUT,), lambda r: (0,)),
      ],
      out_specs=pl.BlockSpec((G, OUT), lambda r: (0, 0)),
      out_shape=jax.ShapeDtypeStruct((G, OUT), _f32),
      scratch_shapes=[
          pltpu.VMEM((G, DIM), _f32),
          pltpu.VMEM((G, HALF), _f32),
      ],
  )(acc, acc, g3, g3, deg2, b3, batch_p, lW1, lb1, lW2, lb2)


def kernel(x, edge_index, batch, W1, b1, W2, b2, W3, b3, lW1, lb1, lW2, lb2):
  src = edge_index[0]
  dst = edge_index[1]
  npad = EPAD - E
  pad_idx = jnp.full((npad,), N, _i32)
  src_p = jnp.concatenate([src.astype(_i32), pad_idx])
  dst_p = jnp.concatenate([dst.astype(_i32), pad_idx])
  src2d = jnp.concatenate([src_p, src_p + NP]).reshape(2 * ER, 128)
  dst2d = dst_p.reshape(ER, 128)
  x_p = jnp.pad(x, ((0, NP - N), (0, 0)))
  batch_p = jnp.concatenate([batch.astype(_i32), jnp.full((NP - N,), G, _i32)])

  deg2 = _sc_degree(dst2d)
  g1 = _tc_g1(x_p, deg2, W1)
  acc1 = _sc_edge_pass(g1, src2d, dst2d)
  g2 = _tc_mid(acc1, g1, deg2, b1, W2)
  acc2 = _sc_edge_pass(g2, src2d, dst2d)
  g3 = _tc_mid(acc2, g2, deg2, b2, W3)
  acc3 = _sc_edge_pass(g3, src2d, dst2d)
  return _tc_final(acc3, g3, deg2, b3, batch_p, lW1, lb1, lW2, lb2)


# speedup vs baseline: 1.0117x; 1.0117x over previous
import functools

import jax
import jax.numpy as jnp
from jax import lax
from jax.experimental import pallas as pl
from jax.experimental.pallas import tpu as pltpu
from jax.experimental.pallas import tpu_sc as plsc

N = 10000
E = 320000
IN = 128
DIM = 256
OUT = 128
G = 64

NP = 10240
EPAD = 327680
ER = EPAD // 128
BR = 256
NRB = NP // BR
HALF = DIM // 2
NT = 16
RPT = NP // NT
EPT = EPAD // NT
NBT = EPT // 128
NBD = EPAD // 32 // 128
CHB = 16

_mesh = plsc.VectorSubcoreMesh(
    core_axis_name="c", subcore_axis_name="s", num_cores=2, num_subcores=NT)

_f32 = jnp.float32
_i32 = jnp.int32


def _zero_vmem2d(ref, nrows):
  z16 = jnp.zeros((16,), _f32)
  @pl.loop(0, nrows)
  def _(i):
    for j in range(8):
      ref[i, pl.ds(j * 16, 16)] = z16


def _sc_degree(dst2d):
  @functools.partial(
      pl.kernel,
      out_type=jax.ShapeDtypeStruct((2, NP), _f32),
      mesh=_mesh,
      scratch_types=[
          pltpu.VMEM((NBD, 128), _i32),
          pltpu.VMEM((128,), _f32),
          pltpu.VMEM((RPT,), _f32),
          pltpu.VMEM_SHARED((NP,), _f32),
      ],
  )
  def deg_kernel(dst_hbm, out_hbm, didx, ones_v, zb, deg_sh):
    c = lax.axis_index("c")
    s = lax.axis_index("s")
    z16 = jnp.zeros((16,), _f32)
    @pl.loop(0, RPT // 16)
    def _(j):
      zb[pl.ds(j * 16, 16)] = z16
    o16 = jnp.ones((16,), _f32)
    @pl.loop(0, 8)
    def _(j):
      ones_v[pl.ds(j * 16, 16)] = o16
    sync = pltpu.sync_copy
    sync(zb, deg_sh.at[pl.ds(s * RPT, RPT)])
    sync(dst_hbm.at[pl.ds((c * NT + s) * NBD, NBD)], didx)
    plsc.subcore_barrier()
    @pl.loop(0, NBD)
    def _(b):
      sync(ones_v, deg_sh.at[didx.at[b]], add=True)
    plsc.subcore_barrier()
    sync(deg_sh.at[pl.ds(s * RPT, RPT)], zb)
    sync(zb, out_hbm.at[c, pl.ds(s * RPT, RPT)])

  return deg_kernel(dst2d)


def _sc_edge_pass(gcat, src2d, dst2d):
  @functools.partial(
      pl.kernel,
      out_type=jax.ShapeDtypeStruct((2 * NP, HALF), _f32),
      mesh=_mesh,
      scratch_types=[
          pltpu.VMEM((CHB, 128), _i32),
          pltpu.VMEM((CHB, 128), _i32),
          pltpu.VMEM((128, HALF), _f32),
          pltpu.VMEM((128, HALF), _f32),
          pltpu.VMEM_SHARED((NP, HALF), _f32),
          pltpu.SemaphoreType.DMA,
          pltpu.SemaphoreType.DMA,
          pltpu.SemaphoreType.DMA,
          pltpu.SemaphoreType.DMA,
      ],
  )
  def edge_kernel(g_hbm, src_hbm, dst_hbm, out_hbm, sidx, didx, rows_a,
                  rows_b, acc_sh, gsem_a, gsem_b, ssem_a, ssem_b):
    c = lax.axis_index("c")
    s = lax.axis_index("s")
    sync = pltpu.sync_copy
    _zero_vmem2d(rows_a, 128)
    @pl.loop(0, 5)
    def _(k):
      sync(rows_a, acc_sh.at[pl.ds(s * RPT + k * 128, 128)])
    plsc.subcore_barrier()
    bufs = (rows_a, rows_b)
    gsems = (gsem_a, gsem_b)
    ssems = (ssem_a, ssem_b)
    @pl.loop(0, NBT // CHB)
    def _(ch):
      sync(src_hbm.at[pl.ds(c * ER + s * NBT + ch * CHB, CHB)], sidx)
      sync(dst_hbm.at[pl.ds(s * NBT + ch * CHB, CHB)], didx)
      pltpu.async_copy(g_hbm.at[sidx.at[0]], bufs[0], gsems[0])
      for b in range(CHB):
        k = b % 2
        if b + 1 < CHB:
          if b >= 1:
            pltpu.make_async_copy(
                bufs[1 - k], acc_sh.at[didx.at[b - 1]], ssems[1 - k]).wait()
          pltpu.async_copy(g_hbm.at[sidx.at[b + 1]], bufs[1 - k], gsems[1 - k])
        pltpu.make_async_copy(g_hbm.at[sidx.at[b]], bufs[k], gsems[k]).wait()
        pltpu.async_copy(bufs[k], acc_sh.at[didx.at[b]], ssems[k], add=True)
      pltpu.make_async_copy(
          bufs[0], acc_sh.at[didx.at[CHB - 2]], ssems[0]).wait()
      pltpu.make_async_copy(
          bufs[1], acc_sh.at[didx.at[CHB - 1]], ssems[1]).wait()
    plsc.subcore_barrier()
    sync(acc_sh.at[pl.ds(s * RPT, RPT)],
         out_hbm.at[pl.ds(c * NP + s * RPT, RPT)])

  return edge_kernel(gcat, src2d, dst2d)


_DN11 = (((1,), (1,)), ((), ()))
_DN00 = (((0,), (0,)), ((), ()))
_PREC = lax.Precision.DEFAULT


def _dinv_of(deg_blk):
  d = deg_blk[0, :] + deg_blk[1, :] + 1.0
  return lax.rsqrt(d).reshape(BR, 1)


def _tc_g1(x_p, deg2, W1):
  def body(x_ref, deg_ref, w_ref, out_ref):
    dinv = _dinv_of(deg_ref[...])
    hw = lax.dot_general(x_ref[...], w_ref[...], _DN11,
                         preferred_element_type=_f32, precision=_PREC)
    out_ref[...] = dinv * hw

  return pl.pallas_call(
      body,
      grid=(2, NRB),
      in_specs=[
          pl.BlockSpec((BR, IN), lambda h, r: (r, 0)),
          pl.BlockSpec((2, BR), lambda h, r: (0, r)),
          pl.BlockSpec((HALF, IN), lambda h, r: (h, 0)),
      ],
      out_specs=pl.BlockSpec((BR, HALF), lambda h, r: (h * NRB + r, 0)),
      out_shape=jax.ShapeDtypeStruct((2 * NP, HALF), _f32),
  )(x_p, deg2, W1)


def _tc_mid(acc, g_prev, deg2, b_prev, W):
  def body(al_ref, ah_ref, gl_ref, gh_ref, deg_ref, b_ref, w_ref, out_ref):
    dinv = _dinv_of(deg_ref[...])
    b = b_ref[...]
    h_lo = jnp.maximum(dinv * (al_ref[...] + gl_ref[...]) + b[:HALF], 0.0)
    h_hi = jnp.maximum(dinv * (ah_ref[...] + gh_ref[...]) + b[HALF:], 0.0)
    w = w_ref[...]
    hw = (lax.dot_general(h_lo, w[:, :HALF], _DN11,
                          preferred_element_type=_f32, precision=_PREC) +
          lax.dot_general(h_hi, w[:, HALF:], _DN11,
                          preferred_element_type=_f32, precision=_PREC))
    out_ref[...] = dinv * hw

  return pl.pallas_call(
      body,
      grid=(2, NRB),
      in_specs=[
          pl.BlockSpec((BR, HALF), lambda h, r: (r, 0)),
          pl.BlockSpec((BR, HALF), lambda h, r: (NRB + r, 0)),
          pl.BlockSpec((BR, HALF), lambda h, r: (r, 0)),
          pl.BlockSpec((BR, HALF), lambda h, r: (NRB + r, 0)),
          pl.BlockSpec((2, BR), lambda h, r: (0, r)),
          pl.BlockSpec((DIM,), lambda h, r: (0,)),
          pl.BlockSpec((HALF, DIM), lambda h, r: (h, 0)),
      ],
      out_specs=pl.BlockSpec((BR, HALF), lambda h, r: (h * NRB + r, 0)),
      out_shape=jax.ShapeDtypeStruct((2 * NP, HALF), _f32),
  )(acc, acc, g_prev, g_prev, deg2, b_prev, W)


def _tc_final(acc, g3, deg2, b3, batch_p, lW1, lb1, lW2, lb2):
  def body(al_ref, ah_ref, gl_ref, gh_ref, deg_ref, b_ref, bat_ref,
           lw1_ref, lb1_ref, lw2_ref, lb2_ref, out_ref, s_acc, c_acc):
    r = pl.program_id(0)
    dinv = _dinv_of(deg_ref[...])
    b = b_ref[...]
    h_lo = dinv * (al_ref[...] + gl_ref[...]) + b[:HALF]
    h_hi = dinv * (ah_ref[...] + gh_ref[...]) + b[HALF:]
    bat = bat_ref[...]
    gids = lax.broadcasted_iota(_i32, (BR, G), 1)
    onehot = (bat.reshape(BR, 1) == gids).astype(_f32)
    s_lo = lax.dot_general(onehot, h_lo, _DN00,
                           preferred_element_type=_f32, precision=_PREC)
    s_hi = lax.dot_general(onehot, h_hi, _DN00,
                           preferred_element_type=_f32, precision=_PREC)
    cnt = jnp.broadcast_to(jnp.sum(onehot, axis=0).reshape(G, 1), (G, HALF))

    @pl.when(r == 0)
    def _():
      s_acc[:, :HALF] = s_lo
      s_acc[:, HALF:] = s_hi
      c_acc[...] = cnt

    @pl.when(r > 0)
    def _():
      s_acc[:, :HALF] += s_lo
      s_acc[:, HALF:] += s_hi
      c_acc[...] += cnt

    @pl.when(r == NRB - 1)
    def _():
      pooled = s_acc[...] / jnp.maximum(c_acc[:, :1], 1.0)
      z = jnp.maximum(
          lax.dot_general(pooled, lw1_ref[...], _DN11,
                          preferred_element_type=_f32, precision=_PREC)
          + lb1_ref[...], 0.0)
      out_ref[...] = lax.dot_general(
          z, lw2_ref[...], _DN11,
          preferred_element_type=_f32, precision=_PREC) + lb2_ref[...]

  return pl.pallas_call(
      body,
      grid=(NRB,),
      in_specs=[
          pl.BlockSpec((BR, HALF), lambda r: (r, 0)),
          pl.BlockSpec((BR, HALF), lambda r: (NRB + r, 0)),
          pl.BlockSpec((BR, HALF), lambda r: (r, 0)),
          pl.BlockSpec((BR, HALF), lambda r: (NRB + r, 0)),
          pl.BlockSpec((2, BR), lambda r: (0, r)),
          pl.BlockSpec((DIM,), lambda r: (0,)),
          pl.BlockSpec((BR,), lambda r: (r,)),
          pl.BlockSpec((DIM, DIM), lambda r: (0, 0)),
          pl.BlockSpec((DIM,), lambda r: (0,)),
          pl.BlockSpec((OUT, DIM), lambda r: (0, 0)),
          pl.BlockSpec((OUT,), lambda r: (0,)),
      ],
      out_specs=pl.BlockSpec((G, OUT), lambda r: (0, 0)),
      out_shape=jax.ShapeDtypeStruct((G, OUT), _f32),
      scratch_shapes=[
          pltpu.VMEM((G, DIM), _f32),
          pltpu.VMEM((G, HALF), _f32),
      ],
  )(acc, acc, g3, g3, deg2, b3, batch_p, lW1, lb1, lW2, lb2)


def kernel(x, edge_index, batch, W1, b1, W2, b2, W3, b3, lW1, lb1, lW2, lb2):
  src = edge_index[0]
  dst = edge_index[1]
  npad = EPAD - E
  pad_idx = jnp.full((npad,), N, _i32)
  src_p = jnp.concatenate([src.astype(_i32), pad_idx])
  dst_p = jnp.concatenate([dst.astype(_i32), pad_idx])
  src2d = jnp.concatenate([src_p, src_p + NP]).reshape(2 * ER, 128)
  dst2d = dst_p.reshape(ER, 128)
  x_p = jnp.pad(x, ((0, NP - N), (0, 0)))
  batch_p = jnp.concatenate([batch.astype(_i32), jnp.full((NP - N,), G, _i32)])

  deg2 = _sc_degree(dst2d)
  g1 = _tc_g1(x_p, deg2, W1)
  acc1 = _sc_edge_pass(g1, src2d, dst2d)
  g2 = _tc_mid(acc1, g1, deg2, b1, W2)
  acc2 = _sc_edge_pass(g2, src2d, dst2d)
  g3 = _tc_mid(acc2, g2, deg2, b2, W3)
  acc3 = _sc_edge_pass(g3, src2d, dst2d)
  return _tc_final(acc3, g3, deg2, b3, batch_p, lW1, lb1, lW2, lb2)


# speedup vs baseline: 1.0510x; 1.0389x over previous
import functools

import jax
import jax.numpy as jnp
from jax import lax
from jax.experimental import pallas as pl
from jax.experimental.pallas import tpu as pltpu
from jax.experimental.pallas import tpu_sc as plsc

N = 10000
E = 320000
IN = 128
DIM = 256
OUT = 128
G = 64

NP = 10240
EPAD = 327680
ER = EPAD // 128
BR = 512
NRB = NP // BR
HALF = DIM // 2
NT = 16
RPT = NP // NT
EPT = EPAD // NT
NBT = EPT // 128
NBD = EPAD // 32 // 128
CHB = 16

_mesh = plsc.VectorSubcoreMesh(
    core_axis_name="c", subcore_axis_name="s", num_cores=2, num_subcores=NT)

_f32 = jnp.float32
_i32 = jnp.int32


def _zero_vmem2d(ref, nrows):
  z16 = jnp.zeros((16,), _f32)
  @pl.loop(0, nrows)
  def _(i):
    for j in range(8):
      ref[i, pl.ds(j * 16, 16)] = z16


def _sc_degree(dst2d):
  @functools.partial(
      pl.kernel,
      out_type=jax.ShapeDtypeStruct((2, NP), _f32),
      mesh=_mesh,
      scratch_types=[
          pltpu.VMEM((NBD, 128), _i32),
          pltpu.VMEM((128,), _f32),
          pltpu.VMEM((RPT,), _f32),
          pltpu.VMEM_SHARED((NP,), _f32),
      ],
  )
  def deg_kernel(dst_hbm, out_hbm, didx, ones_v, zb, deg_sh):
    c = lax.axis_index("c")
    s = lax.axis_index("s")
    z16 = jnp.zeros((16,), _f32)
    @pl.loop(0, RPT // 16)
    def _(j):
      zb[pl.ds(j * 16, 16)] = z16
    o16 = jnp.ones((16,), _f32)
    @pl.loop(0, 8)
    def _(j):
      ones_v[pl.ds(j * 16, 16)] = o16
    sync = pltpu.sync_copy
    sync(zb, deg_sh.at[pl.ds(s * RPT, RPT)])
    sync(dst_hbm.at[pl.ds((c * NT + s) * NBD, NBD)], didx)
    plsc.subcore_barrier()
    @pl.loop(0, NBD)
    def _(b):
      sync(ones_v, deg_sh.at[didx.at[b]], add=True)
    plsc.subcore_barrier()
    sync(deg_sh.at[pl.ds(s * RPT, RPT)], zb)
    sync(zb, out_hbm.at[c, pl.ds(s * RPT, RPT)])

  return deg_kernel(dst2d)


def _sc_edge_pass(gcat, src2d, dst2d):
  @functools.partial(
      pl.kernel,
      out_type=jax.ShapeDtypeStruct((2 * NP, HALF), _f32),
      mesh=_mesh,
      scratch_types=[
          pltpu.VMEM((CHB, 128), _i32),
          pltpu.VMEM((CHB, 128), _i32),
          pltpu.VMEM((128, HALF), _f32),
          pltpu.VMEM((128, HALF), _f32),
          pltpu.VMEM_SHARED((NP, HALF), _f32),
          pltpu.SemaphoreType.DMA,
          pltpu.SemaphoreType.DMA,
          pltpu.SemaphoreType.DMA,
          pltpu.SemaphoreType.DMA,
      ],
  )
  def edge_kernel(g_hbm, src_hbm, dst_hbm, out_hbm, sidx, didx, rows_a,
                  rows_b, acc_sh, gsem_a, gsem_b, ssem_a, ssem_b):
    c = lax.axis_index("c")
    s = lax.axis_index("s")
    sync = pltpu.sync_copy
    _zero_vmem2d(rows_a, 128)
    @pl.loop(0, 5)
    def _(k):
      sync(rows_a, acc_sh.at[pl.ds(s * RPT + k * 128, 128)])
    plsc.subcore_barrier()
    bufs = (rows_a, rows_b)
    gsems = (gsem_a, gsem_b)
    ssems = (ssem_a, ssem_b)
    @pl.loop(0, NBT // CHB)
    def _(ch):
      sync(src_hbm.at[pl.ds(c * ER + s * NBT + ch * CHB, CHB)], sidx)
      sync(dst_hbm.at[pl.ds(s * NBT + ch * CHB, CHB)], didx)
      pltpu.async_copy(g_hbm.at[sidx.at[0]], bufs[0], gsems[0])
      for b in range(CHB):
        k = b % 2
        if b + 1 < CHB:
          if b >= 1:
            pltpu.make_async_copy(
                bufs[1 - k], acc_sh.at[didx.at[b - 1]], ssems[1 - k]).wait()
          pltpu.async_copy(g_hbm.at[sidx.at[b + 1]], bufs[1 - k], gsems[1 - k])
        pltpu.make_async_copy(g_hbm.at[sidx.at[b]], bufs[k], gsems[k]).wait()
        pltpu.async_copy(bufs[k], acc_sh.at[didx.at[b]], ssems[k], add=True)
      pltpu.make_async_copy(
          bufs[0], acc_sh.at[didx.at[CHB - 2]], ssems[0]).wait()
      pltpu.make_async_copy(
          bufs[1], acc_sh.at[didx.at[CHB - 1]], ssems[1]).wait()
    plsc.subcore_barrier()
    sync(acc_sh.at[pl.ds(s * RPT, RPT)],
         out_hbm.at[pl.ds(c * NP + s * RPT, RPT)])

  return edge_kernel(gcat, src2d, dst2d)


_DN11 = (((1,), (1,)), ((), ()))
_DN00 = (((0,), (0,)), ((), ()))
_PREC = lax.Precision.DEFAULT


def _dinv_of(deg_blk):
  d = deg_blk[0, :] + deg_blk[1, :] + 1.0
  return lax.rsqrt(d).reshape(BR, 1)


def _tc_g1(x_p, deg2, W1):
  def body(x_ref, deg_ref, w_ref, out_ref):
    dinv = _dinv_of(deg_ref[...])
    hw = lax.dot_general(x_ref[...], w_ref[...], _DN11,
                         preferred_element_type=_f32, precision=_PREC)
    out_ref[...] = dinv * hw

  return pl.pallas_call(
      body,
      grid=(2, NRB),
      in_specs=[
          pl.BlockSpec((BR, IN), lambda h, r: (r, 0)),
          pl.BlockSpec((2, BR), lambda h, r: (0, r)),
          pl.BlockSpec((HALF, IN), lambda h, r: (h, 0)),
      ],
      out_specs=pl.BlockSpec((BR, HALF), lambda h, r: (h * NRB + r, 0)),
      out_shape=jax.ShapeDtypeStruct((2 * NP, HALF), _f32),
  )(x_p, deg2, W1)


def _tc_mid(acc, g_prev, deg2, b_prev, W):
  def body(al_ref, ah_ref, gl_ref, gh_ref, deg_ref, b_ref, w_ref, out_ref):
    dinv = _dinv_of(deg_ref[...])
    b = b_ref[...]
    h_lo = jnp.maximum(dinv * (al_ref[...] + gl_ref[...]) + b[:HALF], 0.0)
    h_hi = jnp.maximum(dinv * (ah_ref[...] + gh_ref[...]) + b[HALF:], 0.0)
    w = w_ref[...]
    hw = (lax.dot_general(h_lo, w[:, :HALF], _DN11,
                          preferred_element_type=_f32, precision=_PREC) +
          lax.dot_general(h_hi, w[:, HALF:], _DN11,
                          preferred_element_type=_f32, precision=_PREC))
    out_ref[...] = dinv * hw

  return pl.pallas_call(
      body,
      grid=(2, NRB),
      in_specs=[
          pl.BlockSpec((BR, HALF), lambda h, r: (r, 0)),
          pl.BlockSpec((BR, HALF), lambda h, r: (NRB + r, 0)),
          pl.BlockSpec((BR, HALF), lambda h, r: (r, 0)),
          pl.BlockSpec((BR, HALF), lambda h, r: (NRB + r, 0)),
          pl.BlockSpec((2, BR), lambda h, r: (0, r)),
          pl.BlockSpec((DIM,), lambda h, r: (0,)),
          pl.BlockSpec((HALF, DIM), lambda h, r: (h, 0)),
      ],
      out_specs=pl.BlockSpec((BR, HALF), lambda h, r: (h * NRB + r, 0)),
      out_shape=jax.ShapeDtypeStruct((2 * NP, HALF), _f32),
  )(acc, acc, g_prev, g_prev, deg2, b_prev, W)


def _tc_final(acc, g3, deg2, b3, batch_p, lW1, lb1, lW2, lb2):
  def body(al_ref, ah_ref, gl_ref, gh_ref, deg_ref, b_ref, bat_ref,
           lw1_ref, lb1_ref, lw2_ref, lb2_ref, out_ref, s_acc, c_acc):
    r = pl.program_id(0)
    dinv = _dinv_of(deg_ref[...])
    b = b_ref[...]
    h_lo = dinv * (al_ref[...] + gl_ref[...]) + b[:HALF]
    h_hi = dinv * (ah_ref[...] + gh_ref[...]) + b[HALF:]
    bat = bat_ref[...]
    gids = lax.broadcasted_iota(_i32, (BR, G), 1)
    onehot = (bat.reshape(BR, 1) == gids).astype(_f32)
    s_lo = lax.dot_general(onehot, h_lo, _DN00,
                           preferred_element_type=_f32, precision=_PREC)
    s_hi = lax.dot_general(onehot, h_hi, _DN00,
                           preferred_element_type=_f32, precision=_PREC)
    cnt = jnp.broadcast_to(jnp.sum(onehot, axis=0).reshape(G, 1), (G, HALF))

    @pl.when(r == 0)
    def _():
      s_acc[:, :HALF] = s_lo
      s_acc[:, HALF:] = s_hi
      c_acc[...] = cnt

    @pl.when(r > 0)
    def _():
      s_acc[:, :HALF] += s_lo
      s_acc[:, HALF:] += s_hi
      c_acc[...] += cnt

    @pl.when(r == NRB - 1)
    def _():
      pooled = s_acc[...] / jnp.maximum(c_acc[:, :1], 1.0)
      z = jnp.maximum(
          lax.dot_general(pooled, lw1_ref[...], _DN11,
                          preferred_element_type=_f32, precision=_PREC)
          + lb1_ref[...], 0.0)
      out_ref[...] = lax.dot_general(
          z, lw2_ref[...], _DN11,
          preferred_element_type=_f32, precision=_PREC) + lb2_ref[...]

  return pl.pallas_call(
      body,
      grid=(NRB,),
      in_specs=[
          pl.BlockSpec((BR, HALF), lambda r: (r, 0)),
          pl.BlockSpec((BR, HALF), lambda r: (NRB + r, 0)),
          pl.BlockSpec((BR, HALF), lambda r: (r, 0)),
          pl.BlockSpec((BR, HALF), lambda r: (NRB + r, 0)),
          pl.BlockSpec((2, BR), lambda r: (0, r)),
          pl.BlockSpec((DIM,), lambda r: (0,)),
          pl.BlockSpec((BR,), lambda r: (r,)),
          pl.BlockSpec((DIM, DIM), lambda r: (0, 0)),
          pl.BlockSpec((DIM,), lambda r: (0,)),
          pl.BlockSpec((OUT, DIM), lambda r: (0, 0)),
          pl.BlockSpec((OUT,), lambda r: (0,)),
      ],
      out_specs=pl.BlockSpec((G, OUT), lambda r: (0, 0)),
      out_shape=jax.ShapeDtypeStruct((G, OUT), _f32),
      scratch_shapes=[
          pltpu.VMEM((G, DIM), _f32),
          pltpu.VMEM((G, HALF), _f32),
      ],
  )(acc, acc, g3, g3, deg2, b3, batch_p, lW1, lb1, lW2, lb2)


def kernel(x, edge_index, batch, W1, b1, W2, b2, W3, b3, lW1, lb1, lW2, lb2):
  src = edge_index[0]
  dst = edge_index[1]
  npad = EPAD - E
  pad_idx = jnp.full((npad,), N, _i32)
  src_p = jnp.concatenate([src.astype(_i32), pad_idx])
  dst_p = jnp.concatenate([dst.astype(_i32), pad_idx])
  src2d = jnp.concatenate([src_p, src_p + NP]).reshape(2 * ER, 128)
  dst2d = dst_p.reshape(ER, 128)
  x_p = jnp.pad(x, ((0, NP - N), (0, 0)))
  batch_p = jnp.concatenate([batch.astype(_i32), jnp.full((NP - N,), G, _i32)])

  deg2 = _sc_degree(dst2d)
  g1 = _tc_g1(x_p, deg2, W1)
  acc1 = _sc_edge_pass(g1, src2d, dst2d)
  g2 = _tc_mid(acc1, g1, deg2, b1, W2)
  acc2 = _sc_edge_pass(g2, src2d, dst2d)
  g3 = _tc_mid(acc2, g2, deg2, b2, W3)
  acc3 = _sc_edge_pass(g3, src2d, dst2d)
  return _tc_final(acc3, g3, deg2, b3, batch_p, lW1, lb1, lW2, lb2)


# speedup vs baseline: 1.0694x; 1.0175x over previous
import functools

import jax
import jax.numpy as jnp
from jax import lax
from jax.experimental import pallas as pl
from jax.experimental.pallas import tpu as pltpu
from jax.experimental.pallas import tpu_sc as plsc

N = 10000
E = 320000
IN = 128
DIM = 256
OUT = 128
G = 64

NP = 10240
EPAD = 327680
ER = EPAD // 128
BR = 1024
NRB = NP // BR
HALF = DIM // 2
NT = 16
RPT = NP // NT
EPT = EPAD // NT
NBT = EPT // 128
NBD = EPAD // 32 // 128
CHB = 16

_mesh = plsc.VectorSubcoreMesh(
    core_axis_name="c", subcore_axis_name="s", num_cores=2, num_subcores=NT)

_f32 = jnp.float32
_i32 = jnp.int32


def _zero_vmem2d(ref, nrows):
  z16 = jnp.zeros((16,), _f32)
  @pl.loop(0, nrows)
  def _(i):
    for j in range(8):
      ref[i, pl.ds(j * 16, 16)] = z16


def _sc_degree(dst2d):
  @functools.partial(
      pl.kernel,
      out_type=jax.ShapeDtypeStruct((2, NP), _f32),
      mesh=_mesh,
      scratch_types=[
          pltpu.VMEM((NBD, 128), _i32),
          pltpu.VMEM((128,), _f32),
          pltpu.VMEM((RPT,), _f32),
          pltpu.VMEM_SHARED((NP,), _f32),
      ],
  )
  def deg_kernel(dst_hbm, out_hbm, didx, ones_v, zb, deg_sh):
    c = lax.axis_index("c")
    s = lax.axis_index("s")
    z16 = jnp.zeros((16,), _f32)
    @pl.loop(0, RPT // 16)
    def _(j):
      zb[pl.ds(j * 16, 16)] = z16
    o16 = jnp.ones((16,), _f32)
    @pl.loop(0, 8)
    def _(j):
      ones_v[pl.ds(j * 16, 16)] = o16
    sync = pltpu.sync_copy
    sync(zb, deg_sh.at[pl.ds(s * RPT, RPT)])
    sync(dst_hbm.at[pl.ds((c * NT + s) * NBD, NBD)], didx)
    plsc.subcore_barrier()
    @pl.loop(0, NBD)
    def _(b):
      sync(ones_v, deg_sh.at[didx.at[b]], add=True)
    plsc.subcore_barrier()
    sync(deg_sh.at[pl.ds(s * RPT, RPT)], zb)
    sync(zb, out_hbm.at[c, pl.ds(s * RPT, RPT)])

  return deg_kernel(dst2d)


def _sc_edge_pass(gcat, src2d, dst2d):
  @functools.partial(
      pl.kernel,
      out_type=jax.ShapeDtypeStruct((2 * NP, HALF), _f32),
      mesh=_mesh,
      scratch_types=[
          pltpu.VMEM((CHB, 128), _i32),
          pltpu.VMEM((CHB, 128), _i32),
          pltpu.VMEM((128, HALF), _f32),
          pltpu.VMEM((128, HALF), _f32),
          pltpu.VMEM_SHARED((NP, HALF), _f32),
          pltpu.SemaphoreType.DMA,
          pltpu.SemaphoreType.DMA,
          pltpu.SemaphoreType.DMA,
          pltpu.SemaphoreType.DMA,
      ],
  )
  def edge_kernel(g_hbm, src_hbm, dst_hbm, out_hbm, sidx, didx, rows_a,
                  rows_b, acc_sh, gsem_a, gsem_b, ssem_a, ssem_b):
    c = lax.axis_index("c")
    s = lax.axis_index("s")
    sync = pltpu.sync_copy
    _zero_vmem2d(rows_a, 128)
    @pl.loop(0, 5)
    def _(k):
      sync(rows_a, acc_sh.at[pl.ds(s * RPT + k * 128, 128)])
    plsc.subcore_barrier()
    bufs = (rows_a, rows_b)
    gsems = (gsem_a, gsem_b)
    ssems = (ssem_a, ssem_b)
    @pl.loop(0, NBT // CHB)
    def _(ch):
      sync(src_hbm.at[pl.ds(c * ER + s * NBT + ch * CHB, CHB)], sidx)
      sync(dst_hbm.at[pl.ds(s * NBT + ch * CHB, CHB)], didx)
      pltpu.async_copy(g_hbm.at[sidx.at[0]], bufs[0], gsems[0])
      for b in range(CHB):
        k = b % 2
        if b + 1 < CHB:
          if b >= 1:
            pltpu.make_async_copy(
                bufs[1 - k], acc_sh.at[didx.at[b - 1]], ssems[1 - k]).wait()
          pltpu.async_copy(g_hbm.at[sidx.at[b + 1]], bufs[1 - k], gsems[1 - k])
        pltpu.make_async_copy(g_hbm.at[sidx.at[b]], bufs[k], gsems[k]).wait()
        pltpu.async_copy(bufs[k], acc_sh.at[didx.at[b]], ssems[k], add=True)
      pltpu.make_async_copy(
          bufs[0], acc_sh.at[didx.at[CHB - 2]], ssems[0]).wait()
      pltpu.make_async_copy(
          bufs[1], acc_sh.at[didx.at[CHB - 1]], ssems[1]).wait()
    plsc.subcore_barrier()
    sync(acc_sh.at[pl.ds(s * RPT, RPT)],
         out_hbm.at[pl.ds(c * NP + s * RPT, RPT)])

  return edge_kernel(gcat, src2d, dst2d)


_DN11 = (((1,), (1,)), ((), ()))
_DN00 = (((0,), (0,)), ((), ()))
_PREC = lax.Precision.DEFAULT


def _dinv_of(deg_blk):
  d = deg_blk[0, :] + deg_blk[1, :] + 1.0
  return lax.rsqrt(d).reshape(BR, 1)


def _tc_g1(x_p, deg2, W1):
  def body(x_ref, deg_ref, w_ref, out_ref):
    dinv = _dinv_of(deg_ref[...])
    hw = lax.dot_general(x_ref[...], w_ref[...], _DN11,
                         preferred_element_type=_f32, precision=_PREC)
    out_ref[...] = dinv * hw

  return pl.pallas_call(
      body,
      grid=(2, NRB),
      in_specs=[
          pl.BlockSpec((BR, IN), lambda h, r: (r, 0)),
          pl.BlockSpec((2, BR), lambda h, r: (0, r)),
          pl.BlockSpec((HALF, IN), lambda h, r: (h, 0)),
      ],
      out_specs=pl.BlockSpec((BR, HALF), lambda h, r: (h * NRB + r, 0)),
      out_shape=jax.ShapeDtypeStruct((2 * NP, HALF), _f32),
  )(x_p, deg2, W1)


def _tc_mid(acc, g_prev, deg2, b_prev, W):
  def body(al_ref, ah_ref, gl_ref, gh_ref, deg_ref, b_ref, w_ref, out_ref):
    dinv = _dinv_of(deg_ref[...])
    b = b_ref[...]
    h_lo = jnp.maximum(dinv * (al_ref[...] + gl_ref[...]) + b[:HALF], 0.0)
    h_hi = jnp.maximum(dinv * (ah_ref[...] + gh_ref[...]) + b[HALF:], 0.0)
    w = w_ref[...]
    hw = (lax.dot_general(h_lo, w[:, :HALF], _DN11,
                          preferred_element_type=_f32, precision=_PREC) +
          lax.dot_general(h_hi, w[:, HALF:], _DN11,
                          preferred_element_type=_f32, precision=_PREC))
    out_ref[...] = dinv * hw

  return pl.pallas_call(
      body,
      grid=(2, NRB),
      in_specs=[
          pl.BlockSpec((BR, HALF), lambda h, r: (r, 0)),
          pl.BlockSpec((BR, HALF), lambda h, r: (NRB + r, 0)),
          pl.BlockSpec((BR, HALF), lambda h, r: (r, 0)),
          pl.BlockSpec((BR, HALF), lambda h, r: (NRB + r, 0)),
          pl.BlockSpec((2, BR), lambda h, r: (0, r)),
          pl.BlockSpec((DIM,), lambda h, r: (0,)),
          pl.BlockSpec((HALF, DIM), lambda h, r: (h, 0)),
      ],
      out_specs=pl.BlockSpec((BR, HALF), lambda h, r: (h * NRB + r, 0)),
      out_shape=jax.ShapeDtypeStruct((2 * NP, HALF), _f32),
  )(acc, acc, g_prev, g_prev, deg2, b_prev, W)


def _tc_final(acc, g3, deg2, b3, batch_p, lW1, lb1, lW2, lb2):
  def body(al_ref, ah_ref, gl_ref, gh_ref, deg_ref, b_ref, bat_ref,
           lw1_ref, lb1_ref, lw2_ref, lb2_ref, out_ref, s_acc, c_acc):
    r = pl.program_id(0)
    dinv = _dinv_of(deg_ref[...])
    b = b_ref[...]
    h_lo = dinv * (al_ref[...] + gl_ref[...]) + b[:HALF]
    h_hi = dinv * (ah_ref[...] + gh_ref[...]) + b[HALF:]
    bat = bat_ref[...]
    gids = lax.broadcasted_iota(_i32, (BR, G), 1)
    onehot = (bat.reshape(BR, 1) == gids).astype(_f32)
    s_lo = lax.dot_general(onehot, h_lo, _DN00,
                           preferred_element_type=_f32, precision=_PREC)
    s_hi = lax.dot_general(onehot, h_hi, _DN00,
                           preferred_element_type=_f32, precision=_PREC)
    cnt = jnp.broadcast_to(jnp.sum(onehot, axis=0).reshape(G, 1), (G, HALF))

    @pl.when(r == 0)
    def _():
      s_acc[:, :HALF] = s_lo
      s_acc[:, HALF:] = s_hi
      c_acc[...] = cnt

    @pl.when(r > 0)
    def _():
      s_acc[:, :HALF] += s_lo
      s_acc[:, HALF:] += s_hi
      c_acc[...] += cnt

    @pl.when(r == NRB - 1)
    def _():
      pooled = s_acc[...] / jnp.maximum(c_acc[:, :1], 1.0)
      z = jnp.maximum(
          lax.dot_general(pooled, lw1_ref[...], _DN11,
                          preferred_element_type=_f32, precision=_PREC)
          + lb1_ref[...], 0.0)
      out_ref[...] = lax.dot_general(
          z, lw2_ref[...], _DN11,
          preferred_element_type=_f32, precision=_PREC) + lb2_ref[...]

  return pl.pallas_call(
      body,
      grid=(NRB,),
      in_specs=[
          pl.BlockSpec((BR, HALF), lambda r: (r, 0)),
          pl.BlockSpec((BR, HALF), lambda r: (NRB + r, 0)),
          pl.BlockSpec((BR, HALF), lambda r: (r, 0)),
          pl.BlockSpec((BR, HALF), lambda r: (NRB + r, 0)),
          pl.BlockSpec((2, BR), lambda r: (0, r)),
          pl.BlockSpec((DIM,), lambda r: (0,)),
          pl.BlockSpec((BR,), lambda r: (r,)),
          pl.BlockSpec((DIM, DIM), lambda r: (0, 0)),
          pl.BlockSpec((DIM,), lambda r: (0,)),
          pl.BlockSpec((OUT, DIM), lambda r: (0, 0)),
          pl.BlockSpec((OUT,), lambda r: (0,)),
      ],
      out_specs=pl.BlockSpec((G, OUT), lambda r: (0, 0)),
      out_shape=jax.ShapeDtypeStruct((G, OUT), _f32),
      scratch_shapes=[
          pltpu.VMEM((G, DIM), _f32),
          pltpu.VMEM((G, HALF), _f32),
      ],
  )(acc, acc, g3, g3, deg2, b3, batch_p, lW1, lb1, lW2, lb2)


def kernel(x, edge_index, batch, W1, b1, W2, b2, W3, b3, lW1, lb1, lW2, lb2):
  src = edge_index[0]
  dst = edge_index[1]
  npad = EPAD - E
  pad_idx = jnp.full((npad,), N, _i32)
  src_p = jnp.concatenate([src.astype(_i32), pad_idx])
  dst_p = jnp.concatenate([dst.astype(_i32), pad_idx])
  src2d = jnp.concatenate([src_p, src_p + NP]).reshape(2 * ER, 128)
  dst2d = dst_p.reshape(ER, 128)
  x_p = jnp.pad(x, ((0, NP - N), (0, 0)))
  batch_p = jnp.concatenate([batch.astype(_i32), jnp.full((NP - N,), G, _i32)])

  deg2 = _sc_degree(dst2d)
  g1 = _tc_g1(x_p, deg2, W1)
  acc1 = _sc_edge_pass(g1, src2d, dst2d)
  g2 = _tc_mid(acc1, g1, deg2, b1, W2)
  acc2 = _sc_edge_pass(g2, src2d, dst2d)
  g3 = _tc_mid(acc2, g2, deg2, b2, W3)
  acc3 = _sc_edge_pass(g3, src2d, dst2d)
  return _tc_final(acc3, g3, deg2, b3, batch_p, lW1, lb1, lW2, lb2)


# speedup vs baseline: 1.0802x; 1.0101x over previous
import functools

import jax
import jax.numpy as jnp
from jax import lax
from jax.experimental import pallas as pl
from jax.experimental.pallas import tpu as pltpu
from jax.experimental.pallas import tpu_sc as plsc

N = 10000
E = 320000
IN = 128
DIM = 256
OUT = 128
G = 64

NP = 10240
EPAD = 327680
ER = EPAD // 128
BR = 2048
NRB = NP // BR
HALF = DIM // 2
NT = 16
RPT = NP // NT
EPT = EPAD // NT
NBT = EPT // 128
NBD = EPAD // 32 // 128
CHB = 16

_mesh = plsc.VectorSubcoreMesh(
    core_axis_name="c", subcore_axis_name="s", num_cores=2, num_subcores=NT)

_f32 = jnp.float32
_i32 = jnp.int32


def _zero_vmem2d(ref, nrows):
  z16 = jnp.zeros((16,), _f32)
  @pl.loop(0, nrows)
  def _(i):
    for j in range(8):
      ref[i, pl.ds(j * 16, 16)] = z16


def _sc_degree(dst2d):
  @functools.partial(
      pl.kernel,
      out_type=jax.ShapeDtypeStruct((2, NP), _f32),
      mesh=_mesh,
      scratch_types=[
          pltpu.VMEM((NBD, 128), _i32),
          pltpu.VMEM((128,), _f32),
          pltpu.VMEM((RPT,), _f32),
          pltpu.VMEM_SHARED((NP,), _f32),
      ],
  )
  def deg_kernel(dst_hbm, out_hbm, didx, ones_v, zb, deg_sh):
    c = lax.axis_index("c")
    s = lax.axis_index("s")
    z16 = jnp.zeros((16,), _f32)
    @pl.loop(0, RPT // 16)
    def _(j):
      zb[pl.ds(j * 16, 16)] = z16
    o16 = jnp.ones((16,), _f32)
    @pl.loop(0, 8)
    def _(j):
      ones_v[pl.ds(j * 16, 16)] = o16
    sync = pltpu.sync_copy
    sync(zb, deg_sh.at[pl.ds(s * RPT, RPT)])
    sync(dst_hbm.at[pl.ds((c * NT + s) * NBD, NBD)], didx)
    plsc.subcore_barrier()
    @pl.loop(0, NBD)
    def _(b):
      sync(ones_v, deg_sh.at[didx.at[b]], add=True)
    plsc.subcore_barrier()
    sync(deg_sh.at[pl.ds(s * RPT, RPT)], zb)
    sync(zb, out_hbm.at[c, pl.ds(s * RPT, RPT)])

  return deg_kernel(dst2d)


def _sc_edge_pass(gcat, src2d, dst2d):
  @functools.partial(
      pl.kernel,
      out_type=jax.ShapeDtypeStruct((2 * NP, HALF), _f32),
      mesh=_mesh,
      scratch_types=[
          pltpu.VMEM((CHB, 128), _i32),
          pltpu.VMEM((CHB, 128), _i32),
          pltpu.VMEM((128, HALF), _f32),
          pltpu.VMEM((128, HALF), _f32),
          pltpu.VMEM_SHARED((NP, HALF), _f32),
          pltpu.SemaphoreType.DMA,
          pltpu.SemaphoreType.DMA,
          pltpu.SemaphoreType.DMA,
          pltpu.SemaphoreType.DMA,
      ],
  )
  def edge_kernel(g_hbm, src_hbm, dst_hbm, out_hbm, sidx, didx, rows_a,
                  rows_b, acc_sh, gsem_a, gsem_b, ssem_a, ssem_b):
    c = lax.axis_index("c")
    s = lax.axis_index("s")
    sync = pltpu.sync_copy
    _zero_vmem2d(rows_a, 128)
    @pl.loop(0, 5)
    def _(k):
      sync(rows_a, acc_sh.at[pl.ds(s * RPT + k * 128, 128)])
    plsc.subcore_barrier()
    bufs = (rows_a, rows_b)
    gsems = (gsem_a, gsem_b)
    ssems = (ssem_a, ssem_b)
    @pl.loop(0, NBT // CHB)
    def _(ch):
      sync(src_hbm.at[pl.ds(c * ER + s * NBT + ch * CHB, CHB)], sidx)
      sync(dst_hbm.at[pl.ds(s * NBT + ch * CHB, CHB)], didx)
      pltpu.async_copy(g_hbm.at[sidx.at[0]], bufs[0], gsems[0])
      for b in range(CHB):
        k = b % 2
        if b + 1 < CHB:
          if b >= 1:
            pltpu.make_async_copy(
                bufs[1 - k], acc_sh.at[didx.at[b - 1]], ssems[1 - k]).wait()
          pltpu.async_copy(g_hbm.at[sidx.at[b + 1]], bufs[1 - k], gsems[1 - k])
        pltpu.make_async_copy(g_hbm.at[sidx.at[b]], bufs[k], gsems[k]).wait()
        pltpu.async_copy(bufs[k], acc_sh.at[didx.at[b]], ssems[k], add=True)
      pltpu.make_async_copy(
          bufs[0], acc_sh.at[didx.at[CHB - 2]], ssems[0]).wait()
      pltpu.make_async_copy(
          bufs[1], acc_sh.at[didx.at[CHB - 1]], ssems[1]).wait()
    plsc.subcore_barrier()
    sync(acc_sh.at[pl.ds(s * RPT, RPT)],
         out_hbm.at[pl.ds(c * NP + s * RPT, RPT)])

  return edge_kernel(gcat, src2d, dst2d)


_DN11 = (((1,), (1,)), ((), ()))
_DN00 = (((0,), (0,)), ((), ()))
_PREC = lax.Precision.DEFAULT


def _dinv_of(deg_blk):
  d = deg_blk[0, :] + deg_blk[1, :] + 1.0
  return lax.rsqrt(d).reshape(BR, 1)


def _tc_g1(x_p, deg2, W1):
  def body(x_ref, deg_ref, w_ref, out_ref):
    dinv = _dinv_of(deg_ref[...])
    hw = lax.dot_general(x_ref[...], w_ref[...], _DN11,
                         preferred_element_type=_f32, precision=_PREC)
    out_ref[...] = dinv * hw

  return pl.pallas_call(
      body,
      grid=(2, NRB),
      in_specs=[
          pl.BlockSpec((BR, IN), lambda h, r: (r, 0)),
          pl.BlockSpec((2, BR), lambda h, r: (0, r)),
          pl.BlockSpec((HALF, IN), lambda h, r: (h, 0)),
      ],
      out_specs=pl.BlockSpec((BR, HALF), lambda h, r: (h * NRB + r, 0)),
      out_shape=jax.ShapeDtypeStruct((2 * NP, HALF), _f32),
  )(x_p, deg2, W1)


def _tc_mid(acc, g_prev, deg2, b_prev, W):
  def body(al_ref, ah_ref, gl_ref, gh_ref, deg_ref, b_ref, w_ref, out_ref):
    dinv = _dinv_of(deg_ref[...])
    b = b_ref[...]
    h_lo = jnp.maximum(dinv * (al_ref[...] + gl_ref[...]) + b[:HALF], 0.0)
    h_hi = jnp.maximum(dinv * (ah_ref[...] + gh_ref[...]) + b[HALF:], 0.0)
    w = w_ref[...]
    hw = (lax.dot_general(h_lo, w[:, :HALF], _DN11,
                          preferred_element_type=_f32, precision=_PREC) +
          lax.dot_general(h_hi, w[:, HALF:], _DN11,
                          preferred_element_type=_f32, precision=_PREC))
    out_ref[...] = dinv * hw

  return pl.pallas_call(
      body,
      grid=(2, NRB),
      in_specs=[
          pl.BlockSpec((BR, HALF), lambda h, r: (r, 0)),
          pl.BlockSpec((BR, HALF), lambda h, r: (NRB + r, 0)),
          pl.BlockSpec((BR, HALF), lambda h, r: (r, 0)),
          pl.BlockSpec((BR, HALF), lambda h, r: (NRB + r, 0)),
          pl.BlockSpec((2, BR), lambda h, r: (0, r)),
          pl.BlockSpec((DIM,), lambda h, r: (0,)),
          pl.BlockSpec((HALF, DIM), lambda h, r: (h, 0)),
      ],
      out_specs=pl.BlockSpec((BR, HALF), lambda h, r: (h * NRB + r, 0)),
      out_shape=jax.ShapeDtypeStruct((2 * NP, HALF), _f32),
  )(acc, acc, g_prev, g_prev, deg2, b_prev, W)


def _tc_final(acc, g3, deg2, b3, batch_p, lW1, lb1, lW2, lb2):
  def body(al_ref, ah_ref, gl_ref, gh_ref, deg_ref, b_ref, bat_ref,
           lw1_ref, lb1_ref, lw2_ref, lb2_ref, out_ref, s_acc, c_acc):
    r = pl.program_id(0)
    dinv = _dinv_of(deg_ref[...])
    b = b_ref[...]
    h_lo = dinv * (al_ref[...] + gl_ref[...]) + b[:HALF]
    h_hi = dinv * (ah_ref[...] + gh_ref[...]) + b[HALF:]
    bat = bat_ref[...]
    gids = lax.broadcasted_iota(_i32, (BR, G), 1)
    onehot = (bat.reshape(BR, 1) == gids).astype(_f32)
    s_lo = lax.dot_general(onehot, h_lo, _DN00,
                           preferred_element_type=_f32, precision=_PREC)
    s_hi = lax.dot_general(onehot, h_hi, _DN00,
                           preferred_element_type=_f32, precision=_PREC)
    cnt = jnp.broadcast_to(jnp.sum(onehot, axis=0).reshape(G, 1), (G, HALF))

    @pl.when(r == 0)
    def _():
      s_acc[:, :HALF] = s_lo
      s_acc[:, HALF:] = s_hi
      c_acc[...] = cnt

    @pl.when(r > 0)
    def _():
      s_acc[:, :HALF] += s_lo
      s_acc[:, HALF:] += s_hi
      c_acc[...] += cnt

    @pl.when(r == NRB - 1)
    def _():
      pooled = s_acc[...] / jnp.maximum(c_acc[:, :1], 1.0)
      z = jnp.maximum(
          lax.dot_general(pooled, lw1_ref[...], _DN11,
                          preferred_element_type=_f32, precision=_PREC)
          + lb1_ref[...], 0.0)
      out_ref[...] = lax.dot_general(
          z, lw2_ref[...], _DN11,
          preferred_element_type=_f32, precision=_PREC) + lb2_ref[...]

  return pl.pallas_call(
      body,
      grid=(NRB,),
      in_specs=[
          pl.BlockSpec((BR, HALF), lambda r: (r, 0)),
          pl.BlockSpec((BR, HALF), lambda r: (NRB + r, 0)),
          pl.BlockSpec((BR, HALF), lambda r: (r, 0)),
          pl.BlockSpec((BR, HALF), lambda r: (NRB + r, 0)),
          pl.BlockSpec((2, BR), lambda r: (0, r)),
          pl.BlockSpec((DIM,), lambda r: (0,)),
          pl.BlockSpec((BR,), lambda r: (r,)),
          pl.BlockSpec((DIM, DIM), lambda r: (0, 0)),
          pl.BlockSpec((DIM,), lambda r: (0,)),
          pl.BlockSpec((OUT, DIM), lambda r: (0, 0)),
          pl.BlockSpec((OUT,), lambda r: (0,)),
      ],
      out_specs=pl.BlockSpec((G, OUT), lambda r: (0, 0)),
      out_shape=jax.ShapeDtypeStruct((G, OUT), _f32),
      scratch_shapes=[
          pltpu.VMEM((G, DIM), _f32),
          pltpu.VMEM((G, HALF), _f32),
      ],
  )(acc, acc, g3, g3, deg2, b3, batch_p, lW1, lb1, lW2, lb2)


def kernel(x, edge_index, batch, W1, b1, W2, b2, W3, b3, lW1, lb1, lW2, lb2):
  src = edge_index[0]
  dst = edge_index[1]
  npad = EPAD - E
  pad_idx = jnp.full((npad,), N, _i32)
  src_p = jnp.concatenate([src.astype(_i32), pad_idx])
  dst_p = jnp.concatenate([dst.astype(_i32), pad_idx])
  src2d = jnp.concatenate([src_p, src_p + NP]).reshape(2 * ER, 128)
  dst2d = dst_p.reshape(ER, 128)
  x_p = jnp.pad(x, ((0, NP - N), (0, 0)))
  batch_p = jnp.concatenate([batch.astype(_i32), jnp.full((NP - N,), G, _i32)])

  deg2 = _sc_degree(dst2d)
  g1 = _tc_g1(x_p, deg2, W1)
  acc1 = _sc_edge_pass(g1, src2d, dst2d)
  g2 = _tc_mid(acc1, g1, deg2, b1, W2)
  acc2 = _sc_edge_pass(g2, src2d, dst2d)
  g3 = _tc_mid(acc2, g2, deg2, b2, W3)
  acc3 = _sc_edge_pass(g3, src2d, dst2d)
  return _tc_final(acc3, g3, deg2, b3, batch_p, lW1, lb1, lW2, lb2)
